# Initial kernel scaffold; baseline (speedup 1.0000x reference)
import jax
import jax.numpy as jnp
from jax.experimental import pallas as pl


def kernel(x, routing_weights, expert_indices, w_gate, w_up, w_down):
    raise NotImplementedError("write your pallas kernel here")


# speedup vs baseline: 1.3573x; 1.3573x over previous
import functools

import jax
import jax.numpy as jnp
from jax import lax
from jax.experimental import pallas as pl
from jax.experimental.pallas import tpu as pltpu
from jax.experimental.pallas import tpu_sc as plsc

NW = 32


def _make_gather(n_rows, d_model, chunk):
    mesh = plsc.VectorSubcoreMesh(core_axis_name="c", subcore_axis_name="s")
    rows_per_w = n_rows // NW

    @functools.partial(
        pl.kernel,
        out_type=jax.ShapeDtypeStruct((n_rows, d_model), jnp.float32),
        mesh=mesh,
        scratch_types=[
            pltpu.VMEM((chunk,), jnp.int32),
            pltpu.VMEM((chunk, d_model), jnp.float32),
            pltpu.SemaphoreType.DMA,
        ],
    )
    def gather_k(x_hbm, rows_hbm, out_hbm, idx_v, buf_v, sem):
        wid = lax.axis_index("s") * 2 + lax.axis_index("c")
        base = wid * rows_per_w

        def body(i, carry):
            off = base + i * chunk
            pltpu.sync_copy(rows_hbm.at[pl.ds(off, chunk)], idx_v)
            pltpu.async_copy(x_hbm.at[idx_v], buf_v, sem).wait()
            pltpu.sync_copy(buf_v, out_hbm.at[pl.ds(off, chunk)])
            return carry

        lax.fori_loop(0, rows_per_w // chunk, body, 0)

    return gather_k


def _make_combine(n_tokens, d_model, chunk):
    mesh = plsc.VectorSubcoreMesh(core_axis_name="c", subcore_axis_name="s")
    tok_per_w = n_tokens // NW
    lanes_per_row = d_model // 16

    @functools.partial(
        pl.kernel,
        out_type=jax.ShapeDtypeStruct((n_tokens, d_model), jnp.float32),
        mesh=mesh,
        scratch_types=[
            pltpu.VMEM((chunk,), jnp.int32),
            pltpu.VMEM((chunk,), jnp.int32),
            pltpu.VMEM((chunk, d_model), jnp.float32),
            pltpu.VMEM((chunk, d_model), jnp.float32),
            pltpu.SemaphoreType.DMA,
        ],
    )
    def combine_k(y_hbm, r0_hbm, r1_hbm, out_hbm, i0_v, i1_v, a_v, b_v, sem):
        wid = lax.axis_index("s") * 2 + lax.axis_index("c")
        base = wid * tok_per_w

        def body(i, carry):
            off = base + i * chunk
            pltpu.sync_copy(r0_hbm.at[pl.ds(off, chunk)], i0_v)
            pltpu.sync_copy(r1_hbm.at[pl.ds(off, chunk)], i1_v)
            pltpu.async_copy(y_hbm.at[i0_v], a_v, sem).wait()
            pltpu.async_copy(y_hbm.at[i1_v], b_v, sem).wait()

            def row_add(r, c2):
                def col_add(c, c3):
                    for u in range(8):
                        sl = pl.ds((c * 8 + u) * 16, 16)
                        a_v[r, sl] = a_v[r, sl] + b_v[r, sl]
                    return c3

                lax.fori_loop(0, lanes_per_row // 8, col_add, 0)
                return c2

            lax.fori_loop(0, chunk, row_add, 0)
            pltpu.sync_copy(a_v, out_hbm.at[pl.ds(off, chunk)])
            return carry

        lax.fori_loop(0, tok_per_w // chunk, body, 0)

    return combine_k


def _gmm_body(te_ref, xg_ref, wg_ref, wu_ref, wd_ref, wrow_ref, y_ref):
    j = pl.program_id(1)
    x = xg_ref[...]
    g = jnp.dot(x, wg_ref[0].T, preferred_element_type=jnp.float32)
    u = jnp.dot(x, wu_ref[0].T, preferred_element_type=jnp.float32)
    h = (g * jax.nn.sigmoid(g)) * u
    yj = jnp.dot(h, wd_ref[0].T, preferred_element_type=jnp.float32)
    yj = yj * wrow_ref[0, 0, :][:, None]

    @pl.when(j == 0)
    def _():
        y_ref[...] = jnp.zeros_like(y_ref)

    y_ref[...] += yj


def _make_gmm(n_rows, d_model, d_expert, n_experts, blk_m, blk_n):
    nt = n_rows // blk_m
    nb = d_expert // blk_n
    grid_spec = pltpu.PrefetchScalarGridSpec(
        num_scalar_prefetch=1,
        grid=(nt, nb),
        in_specs=[
            pl.BlockSpec((blk_m, d_model), lambda i, j, te: (i, 0)),
            pl.BlockSpec((1, blk_n, d_model), lambda i, j, te: (te[i], j, 0)),
            pl.BlockSpec((1, blk_n, d_model), lambda i, j, te: (te[i], j, 0)),
            pl.BlockSpec((1, d_model, blk_n), lambda i, j, te: (te[i], 0, j)),
            pl.BlockSpec((1, 1, blk_m), lambda i, j, te: (i, 0, 0)),
        ],
        out_specs=pl.BlockSpec((blk_m, d_model), lambda i, j, te: (i, 0)),
    )
    return pl.pallas_call(
        _gmm_body,
        grid_spec=grid_spec,
        out_shape=jax.ShapeDtypeStruct((n_rows, d_model), jnp.float32),
        compiler_params=pltpu.CompilerParams(
            dimension_semantics=("arbitrary", "arbitrary"),
        ),
    )


def kernel(x, routing_weights, expert_indices, w_gate, w_up, w_down):
    batch, seq_len, d_model = x.shape
    top_k = expert_indices.shape[-1]
    n_experts, d_expert, _ = w_gate.shape
    n_tokens = batch * seq_len
    n_assign = n_tokens * top_k

    blk_m = 512
    blk_n = 512
    n_rows = n_assign + n_experts * blk_m

    x_flat = x.reshape(n_tokens, d_model)
    e_flat = expert_indices.reshape(n_assign).astype(jnp.int32)
    w_flat = routing_weights.reshape(n_assign).astype(jnp.float32)

    order = jnp.argsort(e_flat)
    e_sorted = jnp.take(e_flat, order)
    counts = jnp.bincount(e_flat, length=n_experts)
    starts = jnp.cumsum(counts) - counts
    pc = ((counts + blk_m - 1) // blk_m) * blk_m
    padded_starts = jnp.cumsum(pc) - pc
    p = jnp.arange(n_assign, dtype=jnp.int32)
    row_sorted = (padded_starts[e_sorted] + (p - starts[e_sorted])).astype(jnp.int32)
    token_for_sorted = (order // top_k).astype(jnp.int32)
    token_row = jnp.zeros((n_rows,), jnp.int32).at[row_sorted].set(token_for_sorted)
    w_row = jnp.zeros((n_rows,), jnp.float32).at[row_sorted].set(jnp.take(w_flat, order))
    nt = n_rows // blk_m
    tile_start = jnp.arange(nt, dtype=jnp.int32) * blk_m
    pcum = jnp.cumsum(pc)
    tile_expert = jnp.minimum(
        jnp.searchsorted(pcum, tile_start, side="right"), n_experts - 1
    ).astype(jnp.int32)
    row_by_a = jnp.zeros((n_assign,), jnp.int32).at[order].set(row_sorted)
    r0 = row_by_a[0::top_k]
    r1 = row_by_a[1::top_k]

    xg = _make_gather(n_rows, d_model, chunk=64)(x_flat, token_row)

    w_row3 = w_row.reshape(nt, 1, blk_m)
    y = _make_gmm(n_rows, d_model, d_expert, n_experts, blk_m, blk_n)(
        tile_expert, xg, w_gate, w_up, w_down, w_row3
    )

    out = _make_combine(n_tokens, d_model, chunk=32)(y, r0, r1)
    return out.reshape(batch, seq_len, d_model)
